# baseline (speedup 1.0000x reference)
import jax
import jax.numpy as jnp
from jax.experimental import pallas as pl

DIM = 2048
NUM_EXPERTS = 64
BLOCK_T = 512


def _wquant_kernel(w_ref, wq_ref, wsinv_ref):
    w = w_ref[...]
    ws = 1.0 / jnp.clip(jnp.mean(jnp.abs(w)), 1e-5, None)
    wq_ref[...] = jnp.clip(jnp.round(w * ws), -1.0, 1.0)
    wsinv_ref[...] = jnp.full((1, 1), 1.0 / ws, dtype=jnp.float32)


def _gate_kernel(x_ref, wq_ref, wsinv_ref, b_ref, g_ref, o_ref):
    x = x_ref[...]
    g = g_ref[...]
    var = jnp.mean(x * x, axis=-1, keepdims=True)
    rs = jax.lax.rsqrt(var + 1e-6)
    t = x * g[None, :]
    am = jnp.max(jnp.abs(t), axis=-1, keepdims=True)
    sc = 127.0 / jnp.clip(rs * am, 1e-5, None)
    q = jnp.clip(jnp.round(t * (rs * sc)), -128.0, 127.0)
    acc = jax.lax.dot_general(
        q, wq_ref[...],
        dimension_numbers=(((1,), (1,)), ((), ())),
        preferred_element_type=jnp.float32,
    )
    logits = acc * (wsinv_ref[0, 0] / sc) + b_ref[...][None, :]
    m = jnp.max(logits, axis=-1, keepdims=True)
    e = jnp.exp(logits - m)
    o_ref[...] = e / jnp.sum(e, axis=-1, keepdims=True)


@jax.jit
def kernel(x, W, b, g):
    tokens = x.shape[0]
    wq, wsinv = pl.pallas_call(
        _wquant_kernel,
        out_shape=(
            jax.ShapeDtypeStruct((NUM_EXPERTS, DIM), jnp.float32),
            jax.ShapeDtypeStruct((1, 1), jnp.float32),
        ),
    )(W)
    grid = (tokens // BLOCK_T,)
    return pl.pallas_call(
        _gate_kernel,
        grid=grid,
        in_specs=[
            pl.BlockSpec((BLOCK_T, DIM), lambda i: (i, 0)),
            pl.BlockSpec((NUM_EXPERTS, DIM), lambda i: (0, 0)),
            pl.BlockSpec((1, 1), lambda i: (0, 0)),
            pl.BlockSpec((NUM_EXPERTS,), lambda i: (0,)),
            pl.BlockSpec((DIM,), lambda i: (0,)),
        ],
        out_specs=pl.BlockSpec((BLOCK_T, NUM_EXPERTS), lambda i: (i, 0)),
        out_shape=jax.ShapeDtypeStruct((tokens, NUM_EXPERTS), jnp.float32),
    )(x, wq, wsinv, b, g)


# speedup vs baseline: 1.0019x; 1.0019x over previous
import jax
import jax.numpy as jnp
from jax.experimental import pallas as pl
from jax.experimental.pallas import tpu as pltpu

DIM = 2048
NUM_EXPERTS = 64
BLOCK_T = 512


def _wquant_kernel(w_ref, wq_ref, wsinv_ref):
    w = w_ref[...]
    ws = 1.0 / jnp.clip(jnp.mean(jnp.abs(w)), 1e-5, None)
    wq_ref[...] = jnp.clip(jnp.round(w * ws), -1.0, 1.0)
    wsinv_ref[...] = jnp.full((1, 1), 1.0 / ws, dtype=jnp.float32)


def _gate_kernel(x_ref, wq_ref, wsinv_ref, b_ref, g_ref, o_ref):
    x = x_ref[...]
    g = g_ref[...]
    var = jnp.mean(x * x, axis=-1, keepdims=True)
    rs = jax.lax.rsqrt(var + 1e-6)
    t = x * g[None, :]
    am = jnp.max(jnp.abs(t), axis=-1, keepdims=True)
    sc = 127.0 / jnp.clip(rs * am, 1e-5, None)
    q = jnp.clip(jnp.round(t * (rs * sc)), -128.0, 127.0)
    acc = jax.lax.dot_general(
        q, wq_ref[...],
        dimension_numbers=(((1,), (1,)), ((), ())),
        preferred_element_type=jnp.float32,
    )
    logits = acc * (wsinv_ref[0, 0] / sc) + b_ref[...][None, :]
    m = jnp.max(logits, axis=-1, keepdims=True)
    e = jnp.exp(logits - m)
    o_ref[...] = e / jnp.sum(e, axis=-1, keepdims=True)


@jax.jit
def kernel(x, W, b, g):
    tokens = x.shape[0]
    wq, wsinv = pl.pallas_call(
        _wquant_kernel,
        out_shape=(
            jax.ShapeDtypeStruct((NUM_EXPERTS, DIM), jnp.float32),
            jax.ShapeDtypeStruct((1, 1), jnp.float32),
        ),
    )(W)
    grid = (tokens // BLOCK_T,)
    return pl.pallas_call(
        _gate_kernel,
        grid=grid,
        in_specs=[
            pl.BlockSpec((BLOCK_T, DIM), lambda i: (i, 0)),
            pl.BlockSpec((NUM_EXPERTS, DIM), lambda i: (0, 0)),
            pl.BlockSpec((1, 1), lambda i: (0, 0)),
            pl.BlockSpec((NUM_EXPERTS,), lambda i: (0,)),
            pl.BlockSpec((DIM,), lambda i: (0,)),
        ],
        out_specs=pl.BlockSpec((BLOCK_T, NUM_EXPERTS), lambda i: (i, 0)),
        out_shape=jax.ShapeDtypeStruct((tokens, NUM_EXPERTS), jnp.float32),
        compiler_params=pltpu.CompilerParams(
            dimension_semantics=("parallel",),
        ),
    )(x, wq, wsinv, b, g)


# speedup vs baseline: 1.1192x; 1.1171x over previous
import jax
import jax.numpy as jnp
from jax.experimental import pallas as pl
from jax.experimental.pallas import tpu as pltpu

DIM = 2048
NUM_EXPERTS = 64
BLOCK_T = 1024


def _wquant_kernel(w_ref, wq_ref, wsinv_ref):
    w = w_ref[...]
    ws = 1.0 / jnp.clip(jnp.mean(jnp.abs(w)), 1e-5, None)
    wq_ref[...] = jnp.clip(jnp.round(w * ws), -1.0, 1.0)
    wsinv_ref[...] = jnp.full((1, 1), 1.0 / ws, dtype=jnp.float32)


def _gate_kernel(x_ref, wq_ref, wsinv_ref, b_ref, g_ref, o_ref):
    x = x_ref[...]
    g = g_ref[...]
    var = jnp.mean(x * x, axis=-1, keepdims=True)
    rs = jax.lax.rsqrt(var + 1e-6)
    t = x * g[None, :]
    am = jnp.max(jnp.abs(t), axis=-1, keepdims=True)
    sc = 127.0 / jnp.clip(rs * am, 1e-5, None)
    q = jnp.clip(jnp.round(t * (rs * sc)), -128.0, 127.0)
    acc = jax.lax.dot_general(
        q, wq_ref[...],
        dimension_numbers=(((1,), (1,)), ((), ())),
        preferred_element_type=jnp.float32,
    )
    logits = acc * (wsinv_ref[0, 0] / sc) + b_ref[...][None, :]
    m = jnp.max(logits, axis=-1, keepdims=True)
    e = jnp.exp(logits - m)
    o_ref[...] = e / jnp.sum(e, axis=-1, keepdims=True)


@jax.jit
def kernel(x, W, b, g):
    tokens = x.shape[0]
    wq, wsinv = pl.pallas_call(
        _wquant_kernel,
        out_shape=(
            jax.ShapeDtypeStruct((NUM_EXPERTS, DIM), jnp.float32),
            jax.ShapeDtypeStruct((1, 1), jnp.float32),
        ),
    )(W)
    grid = (tokens // BLOCK_T,)
    return pl.pallas_call(
        _gate_kernel,
        grid=grid,
        in_specs=[
            pl.BlockSpec((BLOCK_T, DIM), lambda i: (i, 0)),
            pl.BlockSpec((NUM_EXPERTS, DIM), lambda i: (0, 0)),
            pl.BlockSpec((1, 1), lambda i: (0, 0)),
            pl.BlockSpec((NUM_EXPERTS,), lambda i: (0,)),
            pl.BlockSpec((DIM,), lambda i: (0,)),
        ],
        out_specs=pl.BlockSpec((BLOCK_T, NUM_EXPERTS), lambda i: (i, 0)),
        out_shape=jax.ShapeDtypeStruct((tokens, NUM_EXPERTS), jnp.float32),
        compiler_params=pltpu.CompilerParams(
            dimension_semantics=("parallel",),
        ),
    )(x, wq, wsinv, b, g)
